# baseline (speedup 1.0000x reference)
import jax
import jax.numpy as jnp
from jax.experimental import pallas as pl


N, D, E = 10000, 128, 160000
B, T, F = 4096, 50, 16


def _elu(x):
    return jnp.where(x > 0, x, jnp.exp(jnp.minimum(x, 0.0)) - 1.0)


def _mlp_body(fused_ref, Wfc1_ref, bfc1_ref, g1_ref, be1_ref, Wfc2_ref,
              bfc2_ref, g2_ref, be2_ref, Wfc3_ref, bfc3_ref, Wout_ref,
              bout_ref, o_ref):
    x = fused_ref[...]

    def bn(v, g, b):
        mu = jnp.mean(v, axis=0, keepdims=True)
        var = jnp.mean((v - mu) ** 2, axis=0, keepdims=True)
        return g * (v - mu) / jnp.sqrt(var + 1e-5) + b

    o = jnp.dot(x, Wfc1_ref[...].T, preferred_element_type=jnp.float32)
    o = _elu(bn(o + bfc1_ref[...], g1_ref[...], be1_ref[...]))
    o = jnp.dot(o, Wfc2_ref[...].T, preferred_element_type=jnp.float32)
    o = _elu(bn(o + bfc2_ref[...], g2_ref[...], be2_ref[...]))
    o = _elu(jnp.dot(o, Wfc3_ref[...].T, preferred_element_type=jnp.float32)
             + bfc3_ref[...])
    o_ref[...] = jnp.sum(o * Wout_ref[...], axis=1, keepdims=True) \
        + bout_ref[...]


def _mlp_head(fused, Wfc1, bfc1, g1, be1, Wfc2, bfc2, g2, be2, Wfc3, bfc3,
              Wout, bout):
    return pl.pallas_call(
        _mlp_body,
        out_shape=jax.ShapeDtypeStruct((B, 1), jnp.float32),
    )(fused, Wfc1, bfc1.reshape(1, -1), g1.reshape(1, -1), be1.reshape(1, -1),
      Wfc2, bfc2.reshape(1, -1), g2.reshape(1, -1), be2.reshape(1, -1),
      Wfc3, bfc3.reshape(1, -1), Wout, bout.reshape(1, -1))


def _gatv2(x, ei, Wl, bl, Wr, br, att, bias, heads, outc, concat):
    n = x.shape[0]
    src, dst = ei[0], ei[1]
    xl = (x @ Wl.T + bl).reshape(n, heads, outc)
    xr = (x @ Wr.T + br).reshape(n, heads, outc)
    e = jax.nn.leaky_relu(xl[src] + xr[dst], 0.2)
    logits = (e * att[None, :, :]).sum(-1)
    m = jax.ops.segment_max(logits, dst, num_segments=n)
    m = jnp.where(jnp.isfinite(m), m, 0.0)
    a = jnp.exp(logits - m[dst])
    denom = jax.ops.segment_sum(a, dst, num_segments=n)
    alpha = a / (denom[dst] + 1e-16)
    out = jax.ops.segment_sum(xl[src] * alpha[:, :, None], dst, num_segments=n)
    out = out.reshape(n, heads * outc) if concat else out.mean(axis=1)
    return out + bias


def _gru_last(seq, lengths, Wih, Whh, bih, bhh):
    b = seq.shape[0]
    h0 = jnp.zeros((b, Whh.shape[1]), seq.dtype)

    def step(h, inp):
        x_t, t = inp
        gi = x_t @ Wih.T + bih
        gh = h @ Whh.T + bhh
        ir, iz, inn = jnp.split(gi, 3, axis=1)
        hr, hz, hn = jnp.split(gh, 3, axis=1)
        r = jax.nn.sigmoid(ir + hr)
        z = jax.nn.sigmoid(iz + hz)
        ng = jnp.tanh(inn + r * hn)
        h_new = (1.0 - z) * ng + z * h
        h = jnp.where((t < lengths)[:, None], h_new, h)
        return h, None

    hT, _ = jax.lax.scan(step, h0, (jnp.swapaxes(seq, 0, 1),
                                    jnp.arange(seq.shape[1])))
    return hT


def kernel(x_nodes, edge_index, seq_batch, lengths, stop_indices, W1l, b1l,
           W1r, b1r, att1, bias1, W2l, b2l, W2r, b2r, att2, bias2, Wih, Whh,
           bih, bhh, Wfc1, bfc1, g1, be1, Wfc2, bfc2, g2, be2, Wfc3, bfc3,
           Wout, bout):
    n = x_nodes.shape[0]
    loops = jnp.arange(n, dtype=edge_index.dtype)
    ei = jnp.concatenate([edge_index, jnp.stack([loops, loops])], axis=1)
    x = _elu(_gatv2(x_nodes, ei, W1l, b1l, W1r, b1r, att1, bias1, 4, 64, True))
    x = _elu(_gatv2(x, ei, W2l, b2l, W2r, b2r, att2, bias2, 1, 128, False))
    stop_emb = x[stop_indices]
    temp_emb = _gru_last(seq_batch, lengths, Wih, Whh, bih, bhh)
    fused = jnp.concatenate([stop_emb, temp_emb], axis=1)
    return _mlp_head(fused, Wfc1, bfc1, g1, be1, Wfc2, bfc2, g2, be2,
                     Wfc3, bfc3, Wout, bout)


# speedup vs baseline: 4.7638x; 4.7638x over previous
import functools

import jax
import jax.numpy as jnp
from jax import lax
from jax.experimental import pallas as pl
from jax.experimental.pallas import tpu as pltpu
from jax.experimental.pallas import tpu_sc as plsc


N, D, E = 10000, 128, 160000
B, T, F = 4096, 50, 16

NP = 10240
EP = 163840
EP3 = EP + NP

_SC_INFO = plsc.get_sparse_core_info()
_NC, _NS, _L = _SC_INFO.num_cores, _SC_INFO.num_subcores, _SC_INFO.num_lanes
_NW = _NC * _NS

_NEG = -3.0e38


def _iota16():
    return lax.iota(jnp.int32, 16)


def _f16(x, dtype=jnp.int32):
    return jnp.full((16,), x, dtype)


def _elu(x):
    return jnp.where(x > 0, x, jnp.exp(jnp.minimum(x, 0.0)) - 1.0)


def _scatter_combine(tab, tmpf, tmpi, dstv, vals, valid, op):
    li = _iota16()
    neutral = jnp.float32(_NEG if op == "max" else 0.0)

    def shift(vec, idx, tmp):
        tmp[...] = vec
        return plsc.load_gather(tmp, [idx])

    vld01 = jnp.where(valid, 1, 0)
    skeys, sperm = plsc.sort_key_val(dstv, li)
    svld = shift(vld01, sperm, tmpi)
    svs = []
    for v in vals:
        sv = shift(v, sperm, tmpf)
        svs.append(jnp.where(svld == 1, sv, neutral))
    for k in (1, 2, 4, 8):
        idx = jnp.maximum(li - k, 0)
        cond = jnp.logical_and(li >= k, shift(skeys, idx, tmpi) == skeys)
        for i, sv in enumerate(svs):
            sh = shift(sv, idx, tmpf)
            comb = jnp.maximum(sv, sh) if op == "max" else sv + sh
            svs[i] = jnp.where(cond, comb, sv)
        svld = jnp.where(cond, jnp.maximum(svld, shift(svld, idx, tmpi)),
                         svld)
    nxt = shift(skeys, jnp.minimum(li + 1, 15), tmpi)
    last = jnp.logical_or(li == 15, nxt != skeys)
    writers = jnp.logical_and(last, svld == 1)
    for h, sv in enumerate(svs):
        idxh = skeys + h * NP
        cur = plsc.load_gather(tab, [idxh], mask=writers)
        nv = jnp.maximum(cur, sv) if op == "max" else cur + sv
        plsc.store_scatter(tab, [idxh], nv, mask=writers)


def _init_vec(ref, n, value):
    def bd(i, _):
        ref[pl.ds(i * 16, 16)] = jnp.full((16,), value, ref.dtype)
        return 0

    lax.fori_loop(0, n // 16, bd, 0)


def _tile_dump(tab, out_hbm, wid, ht):
    pltpu.sync_copy(tab, out_hbm.at[pl.ds(wid * ht, ht)])


def _gat_sweep1(xsl, xsr, src_e, dst_e, att_flat, heads, dim, halves):
    C = dim // heads
    cb = 80 if halves == 2 else 160
    ept = EP // _NW
    nb = ept // cb
    ht = heads * NP
    rs = ht // _NS
    mesh = plsc.VectorSubcoreMesh(core_axis_name="c", subcore_axis_name="s")

    scratch = [
        pltpu.VMEM((dim,), jnp.float32),
        pltpu.VMEM((cb,), jnp.int32),
        pltpu.VMEM((cb,), jnp.int32),
        pltpu.VMEM((cb,), jnp.int32),
        pltpu.VMEM((cb,), jnp.int32),
        pltpu.VMEM((cb, 128), jnp.float32),
        pltpu.VMEM((cb, 128), jnp.float32),
        pltpu.VMEM((cb, 128), jnp.float32),
        pltpu.VMEM((cb, 128), jnp.float32),
        pltpu.VMEM((heads * cb,), jnp.float32),
        pltpu.VMEM((ht,), jnp.float32),
        pltpu.VMEM((16,), jnp.float32),
        pltpu.VMEM((16,), jnp.int32),
    ]
    if halves == 1:
        scratch[6] = pltpu.VMEM((16,), jnp.float32)
        scratch[8] = pltpu.VMEM((16,), jnp.float32)

    @functools.partial(
        pl.kernel, mesh=mesh,
        compiler_params=pltpu.CompilerParams(needs_layout_passes=False),
        out_type=(jax.ShapeDtypeStruct((heads * EP,), jnp.float32),
                  jax.ShapeDtypeStruct((_NW * ht,), jnp.float32)),
        scratch_types=scratch,
    )
    def k(xsl_h, xsr_h, src_h, dst_h, att_h, logits_h, mpart_h,
          att_v, src_v, aux_v, dst_v, aux2_v, bl_lo, bl_hi, br_lo, br_hi,
          lbuf, mtab, tmpf, tmpi):
        cid = lax.axis_index("c")
        sid = lax.axis_index("s")
        wid = sid * _NC + cid
        base = wid * ept
        pltpu.sync_copy(att_h, att_v)
        _init_vec(mtab, ht, _NEG)
        it16 = _iota16()

        def block(b, _):
            bb = base + b * cb
            pltpu.sync_copy(src_h.at[pl.ds(bb, cb)], src_v)
            pltpu.sync_copy(dst_h.at[pl.ds(bb, cb)], dst_v)
            if halves == 2:
                def off(j, _):
                    sl = pl.ds(j * 16, 16)
                    aux_v[sl] = src_v[sl] + NP
                    aux2_v[sl] = dst_v[sl] + NP
                    return 0
                lax.fori_loop(0, cb // 16, off, 0)
            pltpu.sync_copy(xsl_h.at[src_v], bl_lo)
            pltpu.sync_copy(xsr_h.at[dst_v], br_lo)
            if halves == 2:
                pltpu.sync_copy(xsl_h.at[aux_v], bl_hi)
                pltpu.sync_copy(xsr_h.at[aux2_v], br_hi)

            def group(g, _):
                eb = g * 16
                dstv = dst_v[pl.ds(eb, 16)]
                valid = (bb + eb) + it16 < E
                ev = eb + it16
                vals = []
                for h in range(heads):
                    hf = (h * C) // 128
                    bl = bl_lo if hf == 0 else bl_hi
                    br = br_lo if hf == 0 else br_hi
                    f0 = (h * C) % 128

                    def cbody(c, acc, bl=bl, br=br, f0=f0, h=h):
                        fv = _f16(f0 + c)
                        s = (plsc.load_gather(bl, [ev, fv])
                             + plsc.load_gather(br, [ev, fv]))
                        lk = jnp.maximum(s, 0.2 * s)
                        av = plsc.load_gather(att_v, [_f16(h * C + c)])
                        return acc + av * lk

                    accv = lax.fori_loop(0, C, cbody,
                                         jnp.zeros((16,), jnp.float32))
                    lbuf[pl.ds(h * cb + eb, 16)] = accv
                    vals.append(accv)
                _scatter_combine(mtab, tmpf, tmpi, dstv, vals, valid, "max")
                return 0

            lax.fori_loop(0, cb // 16, group, 0)
            for h in range(heads):
                pltpu.sync_copy(lbuf.at[pl.ds(h * cb, cb)],
                                logits_h.at[pl.ds(h * EP + bb, cb)])
            return 0

        lax.fori_loop(0, nb, block, 0)
        _tile_dump(mtab, mpart_h, wid, ht)

    return k(xsl, xsr, src_e, dst_e, att_flat)


def _merge_a(m_part, lself, heads):
    ht = heads * NP
    rows = NP // _NW
    mesh = plsc.VectorSubcoreMesh(core_axis_name="c", subcore_axis_name="s")

    @functools.partial(
        pl.kernel, mesh=mesh,
        compiler_params=pltpu.CompilerParams(needs_layout_passes=False),
        out_type=(jax.ShapeDtypeStruct((ht,), jnp.float32),
                  jax.ShapeDtypeStruct((ht,), jnp.float32)),
        scratch_types=[
            pltpu.VMEM((rows * 8,), jnp.float32),
            pltpu.VMEM((rows,), jnp.float32),
            pltpu.VMEM((rows,), jnp.float32),
            pltpu.VMEM((rows,), jnp.float32),
            pltpu.VMEM((rows,), jnp.float32),
        ],
    )
    def k(mp_h, ls_h, mfin_h, aself_h, ls_v, m0, m1, mf, asb):
        cid = lax.axis_index("c")
        sid = lax.axis_index("s")
        wid = sid * _NC + cid
        r0 = wid * rows
        it16 = _iota16()
        pltpu.sync_copy(ls_h.at[pl.ds(r0 * 8, rows * 8)], ls_v)
        for h in range(heads):
            pltpu.sync_copy(mp_h.at[pl.ds(h * NP + r0, rows)], m0)
            for w in range(1, _NW):
                pltpu.sync_copy(mp_h.at[pl.ds(w * ht + h * NP + r0, rows)],
                                m1)

                def rd(g, _):
                    sl = pl.ds(g * 16, 16)
                    m0[sl] = jnp.maximum(m0[sl], m1[sl])
                    return 0

                lax.fori_loop(0, rows // 16, rd, 0)

            def bd(g, _, h=h):
                sl = pl.ds(g * 16, 16)
                lsv = plsc.load_gather(ls_v, [(g * 16 + it16) * 8 + h])
                mv = jnp.maximum(m0[sl], lsv)
                mf[sl] = mv
                asb[sl] = jnp.exp(lsv - mv)
                return 0

            lax.fori_loop(0, rows // 16, bd, 0)
            pltpu.sync_copy(mf, mfin_h.at[pl.ds(h * NP + r0, rows)])
            pltpu.sync_copy(asb, aself_h.at[pl.ds(h * NP + r0, rows)])

    return k(m_part, lself)


def _gat_sweep2(logits, dst_e, m_fin, heads):
    cb = 320
    ept = EP // _NW
    nb = ept // cb
    ht = heads * NP
    rs = ht // _NS
    mesh = plsc.VectorSubcoreMesh(core_axis_name="c", subcore_axis_name="s")

    @functools.partial(
        pl.kernel, mesh=mesh,
        compiler_params=pltpu.CompilerParams(needs_layout_passes=False),
        out_type=(jax.ShapeDtypeStruct((heads * EP,), jnp.float32),
                  jax.ShapeDtypeStruct((_NW * ht,), jnp.float32)),
        scratch_types=[
            pltpu.VMEM((ht,), jnp.float32),
            pltpu.VMEM((ht,), jnp.float32),
            pltpu.VMEM((cb,), jnp.int32),
            pltpu.VMEM((heads * cb,), jnp.float32),
            pltpu.VMEM((heads * cb,), jnp.float32),
            pltpu.VMEM((16,), jnp.float32),
            pltpu.VMEM((16,), jnp.int32),
        ],
    )
    def k(lg_h, dst_h, mfin_h, a_h, dpart_h,
          m_v, den, dst_v, lbuf, abuf, tmpf, tmpi):
        cid = lax.axis_index("c")
        sid = lax.axis_index("s")
        wid = sid * _NC + cid
        base = wid * ept
        it16 = _iota16()
        pltpu.sync_copy(mfin_h, m_v)
        _init_vec(den, ht, 0.0)

        def block(b, _):
            bb = base + b * cb
            pltpu.sync_copy(dst_h.at[pl.ds(bb, cb)], dst_v)
            for h in range(heads):
                pltpu.sync_copy(lg_h.at[pl.ds(h * EP + bb, cb)],
                                lbuf.at[pl.ds(h * cb, cb)])

            def group(g, _):
                eb = g * 16
                dstv = dst_v[pl.ds(eb, 16)]
                valid = (bb + eb) + it16 < E
                vals = []
                for h in range(heads):
                    mg = plsc.load_gather(m_v, [dstv + h * NP])
                    a = jnp.exp(lbuf[pl.ds(h * cb + eb, 16)] - mg)
                    abuf[pl.ds(h * cb + eb, 16)] = a
                    vals.append(a)
                _scatter_combine(den, tmpf, tmpi, dstv, vals, valid, "add")
                return 0

            lax.fori_loop(0, cb // 16, group, 0)
            for h in range(heads):
                pltpu.sync_copy(abuf.at[pl.ds(h * cb, cb)],
                                a_h.at[pl.ds(h * EP + bb, cb)])
            return 0

        lax.fori_loop(0, nb, block, 0)
        _tile_dump(den, dpart_h, wid, ht)

    return k(logits, dst_e, m_fin)


def _alpha_sweep(a_full, dst_e, dinv, heads):
    cb = 320
    ept = EP3 // _NW
    nb = ept // cb
    ht = heads * NP
    mesh = plsc.VectorSubcoreMesh(core_axis_name="c", subcore_axis_name="s")

    @functools.partial(
        pl.kernel, mesh=mesh,
        compiler_params=pltpu.CompilerParams(needs_layout_passes=False),
        out_type=jax.ShapeDtypeStruct((heads * EP3,), jnp.float32),
        scratch_types=[
            pltpu.VMEM((ht,), jnp.float32),
            pltpu.VMEM((cb,), jnp.int32),
            pltpu.VMEM((heads * cb,), jnp.float32),
            pltpu.VMEM((heads * cb,), jnp.float32),
        ],
    )
    def k(af_h, dst_h, dinv_h, alpha_h, dv, dst_v, abuf, alb):
        cid = lax.axis_index("c")
        sid = lax.axis_index("s")
        wid = sid * _NC + cid
        base = wid * ept
        it16 = _iota16()
        pltpu.sync_copy(dinv_h, dv)

        def block(b, _):
            bb = base + b * cb
            pltpu.sync_copy(dst_h.at[pl.ds(bb, cb)], dst_v)
            for h in range(heads):
                pltpu.sync_copy(af_h.at[pl.ds(h * EP3 + bb, cb)],
                                abuf.at[pl.ds(h * cb, cb)])

            def group(g, _):
                eb = g * 16
                dstv = dst_v[pl.ds(eb, 16)]
                gid = (bb + eb) + it16
                valid = jnp.logical_or(gid < E, gid >= EP)
                for h in range(heads):
                    di = plsc.load_gather(dv, [dstv + h * NP])
                    al = abuf[pl.ds(h * cb + eb, 16)] * di
                    alb[pl.ds(h * cb + eb, 16)] = jnp.where(valid, al, 0.0)
                return 0

            lax.fori_loop(0, cb // 16, group, 0)
            for h in range(heads):
                pltpu.sync_copy(alb.at[pl.ds(h * cb, cb)],
                                alpha_h.at[pl.ds(h * EP3 + bb, cb)])
            return 0

        lax.fori_loop(0, nb, block, 0)

    return k(a_full, dst_e, dinv)


def _gat_sweep3(table, src_e, dst_e, alpha, heads, feature_split):
    cb = 160
    if feature_split:
        ept = EP3 // _NS
    else:
        ept = EP3 // _NW
    nb = ept // cb
    mesh = plsc.VectorSubcoreMesh(core_axis_name="c", subcore_axis_name="s")

    @functools.partial(
        pl.kernel, mesh=mesh,
        compiler_params=pltpu.CompilerParams(needs_layout_passes=False),
        out_type=jax.ShapeDtypeStruct((2 * NP, 128), jnp.float32),
        scratch_types=[
            pltpu.VMEM((cb,), jnp.int32),
            pltpu.VMEM((cb,), jnp.int32),
            pltpu.VMEM((cb,), jnp.int32),
            pltpu.VMEM((cb, 128), jnp.float32),
            pltpu.VMEM((heads * cb,), jnp.float32),
            pltpu.VMEM_SHARED((NP, 128), jnp.float32),
        ],
    )
    def k(tab_h, src_h, dst_h, alpha_h, out_h,
          src_v, tix, dst_v, rows, alb, spm):
        cid = lax.axis_index("c")
        sid = lax.axis_index("s")

        def ze(e, _):
            for ch in range(8):
                rows[e, pl.ds(ch * 16, 16)] = jnp.zeros((16,), jnp.float32)
            return 0
        lax.fori_loop(0, cb, ze, 0)
        for q in range(NP // _NS // cb):
            pltpu.sync_copy(rows, spm.at[pl.ds(sid * (NP // _NS) + q * cb,
                                               cb)])
        plsc.subcore_barrier()

        if feature_split:
            tbase = sid * ept
        else:
            tbase = (sid * _NC + cid) * ept

        def block(b, _):
            eb = tbase + b * cb
            pltpu.sync_copy(src_h.at[pl.ds(eb, cb)], src_v)
            pltpu.sync_copy(dst_h.at[pl.ds(eb, cb)], dst_v)
            for h in range(heads):
                pltpu.sync_copy(alpha_h.at[pl.ds(h * EP3 + eb, cb)],
                                alb.at[pl.ds(h * cb, cb)])
            if feature_split:
                def off(j, _):
                    sl = pl.ds(j * 16, 16)
                    tix[sl] = src_v[sl] + cid * NP
                    return 0
                lax.fori_loop(0, cb // 16, off, 0)
                pltpu.sync_copy(tab_h.at[tix], rows)
            else:
                pltpu.sync_copy(tab_h.at[src_v], rows)

            nheads_half = 2 if (feature_split and heads == 4) else 1
            cph = 8 // nheads_half

            def scale(e, _):
                for hh in range(nheads_half):
                    if feature_split and heads == 4:
                        hrow = cid * 2 + hh
                    else:
                        hrow = 0
                    aspl = plsc.load_gather(alb, [_f16(hrow * cb + e)])
                    for ch in range(hh * cph, (hh + 1) * cph):
                        sl = pl.ds(ch * 16, 16)
                        rows[e, sl] = rows[e, sl] * aspl
                return 0

            lax.fori_loop(0, cb, scale, 0)
            pltpu.sync_copy(rows, spm.at[dst_v], add=True)
            return 0

        lax.fori_loop(0, nb, block, 0)
        plsc.subcore_barrier()
        o = sid * (NP // _NS)
        pltpu.sync_copy(spm.at[pl.ds(o, NP // _NS)],
                        out_h.at[pl.ds(cid * NP + o, NP // _NS)])

    return k(table, src_e, dst_e, alpha)


def _proj_body(x_ref, wl_ref, bl_ref, wr_ref, br_ref, attm_ref,
               xl_ref, xr_ref, ls_ref):
    xb = x_ref[...]
    xl = jnp.dot(xb, wl_ref[...].T, preferred_element_type=jnp.float32) \
        + bl_ref[...]
    xr = jnp.dot(xb, wr_ref[...].T, preferred_element_type=jnp.float32) \
        + br_ref[...]
    s = xl + xr
    lk = jnp.maximum(s, 0.2 * s)
    xl_ref[...] = xl
    xr_ref[...] = xr
    ls_ref[...] = jnp.dot(lk, attm_ref[...],
                          preferred_element_type=jnp.float32)


def _proj(x, Wl, bl, Wr, br, attM, din, dim):
    nblk = 16
    rows = NP // nblk
    return pl.pallas_call(
        _proj_body,
        grid=(nblk,),
        in_specs=[
            pl.BlockSpec((rows, din), lambda i: (i, 0)),
            pl.BlockSpec((dim, din), lambda i: (0, 0)),
            pl.BlockSpec((1, dim), lambda i: (0, 0)),
            pl.BlockSpec((dim, din), lambda i: (0, 0)),
            pl.BlockSpec((1, dim), lambda i: (0, 0)),
            pl.BlockSpec((dim, 8), lambda i: (0, 0)),
        ],
        out_specs=[
            pl.BlockSpec((rows, dim), lambda i: (i, 0)),
            pl.BlockSpec((rows, dim), lambda i: (i, 0)),
            pl.BlockSpec((rows, 8), lambda i: (i, 0)),
        ],
        out_shape=[
            jax.ShapeDtypeStruct((NP, dim), jnp.float32),
            jax.ShapeDtypeStruct((NP, dim), jnp.float32),
            jax.ShapeDtypeStruct((NP, 8), jnp.float32),
        ],
    )(x, Wl, bl.reshape(1, -1), Wr, br.reshape(1, -1), attM)


def _merge_b(d_part, a_self, heads):
    ht = heads * NP

    def body(d_ref, a_ref, o_ref):
        o_ref[...] = 1.0 / (jnp.sum(d_ref[...], axis=0) + a_ref[...] + 1e-16)

    out = pl.pallas_call(
        body,
        out_shape=jax.ShapeDtypeStruct((ht // 128, 128), jnp.float32),
    )(d_part.reshape(_NW, ht // 128, 128), a_self.reshape(ht // 128, 128))
    return out.reshape(ht)


def _post1(acc, bias):
    def body(a0_ref, a1_ref, b_ref, o_ref):
        o_ref[:, 0:128] = _elu(a0_ref[...] + b_ref[:, 0:128])
        o_ref[:, 128:256] = _elu(a1_ref[...] + b_ref[:, 128:256])

    nblk = 16
    rows = NP // nblk
    return pl.pallas_call(
        body,
        grid=(nblk,),
        in_specs=[
            pl.BlockSpec((rows, 128), lambda i: (i, 0)),
            pl.BlockSpec((rows, 128), lambda i: (i, 0)),
            pl.BlockSpec((1, 256), lambda i: (0, 0)),
        ],
        out_specs=pl.BlockSpec((rows, 256), lambda i: (i, 0)),
        out_shape=jax.ShapeDtypeStruct((NP, 256), jnp.float32),
    )(acc[0], acc[1], bias.reshape(1, -1))


def _post2(acc, bias):
    def body(a0_ref, a1_ref, b_ref, o_ref):
        o_ref[...] = _elu(a0_ref[...] + a1_ref[...] + b_ref[...])

    nblk = 16
    rows = NP // nblk
    return pl.pallas_call(
        body,
        grid=(nblk,),
        in_specs=[
            pl.BlockSpec((rows, 128), lambda i: (i, 0)),
            pl.BlockSpec((rows, 128), lambda i: (i, 0)),
            pl.BlockSpec((1, 128), lambda i: (0, 0)),
        ],
        out_specs=pl.BlockSpec((rows, 128), lambda i: (i, 0)),
        out_shape=jax.ShapeDtypeStruct((NP, 128), jnp.float32),
    )(acc[0], acc[1], bias.reshape(1, -1))


def _gat_layer(x, Wl, bl, Wr, br, att, bias, heads, dim, din,
               src12, dst12, src3, dst3):
    C = dim // heads
    attM = jnp.zeros((dim, 8), jnp.float32).at[
        jnp.arange(dim), jnp.arange(dim) // C].set(att.reshape(-1))
    xl, xr, ls = _proj(x, Wl, bl, Wr, br, attM, din, dim)
    if dim == 256:
        xsl = jnp.concatenate([xl[:, :128], xl[:, 128:]], axis=0)
        xsr = jnp.concatenate([xr[:, :128], xr[:, 128:]], axis=0)
        logits, m_part = _gat_sweep1(xsl, xsr, src12, dst12,
                                     att.reshape(-1), heads, dim, 2)
    else:
        xsl, xsr = xl, xr
        logits, m_part = _gat_sweep1(xsl, xsr, src12, dst12,
                                     att.reshape(-1), heads, dim, 1)
    m_fin, a_self = _merge_a(m_part, ls.reshape(-1), heads)
    a_edge, d_part = _gat_sweep2(logits, dst12, m_fin, heads)
    dinv = _merge_b(d_part, a_self, heads)
    a_full = jnp.concatenate([a_edge.reshape(heads, EP),
                              a_self.reshape(heads, NP)], axis=1).reshape(-1)
    alpha = _alpha_sweep(a_full, dst3, dinv, heads)
    acc = _gat_sweep3(xsl, src3, dst3, alpha, heads,
                      feature_split=(dim == 256))
    acc = (acc[:NP], acc[NP:])
    if dim == 256:
        return _post1(acc, bias)
    return _post2(acc, bias)


def _sc_row_gather(table, idx, cols):
    nidx = idx.shape[0]
    per_w = nidx // _NW
    mesh = plsc.VectorSubcoreMesh(core_axis_name="c", subcore_axis_name="s")

    @functools.partial(
        pl.kernel, mesh=mesh,
        compiler_params=pltpu.CompilerParams(needs_layout_passes=False),
        out_type=jax.ShapeDtypeStruct((nidx, cols), jnp.float32),
        scratch_types=[
            pltpu.VMEM((per_w,), jnp.int32),
            pltpu.VMEM((per_w, cols), jnp.float32),
            pltpu.SemaphoreType.DMA,
        ],
    )
    def k(table_hbm, idx_hbm, out_hbm, idx_v, rows_v, sem):
        wid = lax.axis_index("s") * _NC + lax.axis_index("c")
        base = wid * per_w
        pltpu.sync_copy(idx_hbm.at[pl.ds(base, per_w)], idx_v)
        pltpu.async_copy(table_hbm.at[idx_v], rows_v, sem).wait()
        pltpu.sync_copy(rows_v, out_hbm.at[pl.ds(base, per_w)])

    return k(table, idx)


def _mlp_body(se_ref, te_ref, Wfc1_ref, bfc1_ref, g1_ref, be1_ref, Wfc2_ref,
              bfc2_ref, g2_ref, be2_ref, Wfc3_ref, bfc3_ref, Wout_ref,
              bout_ref, o_ref):
    x = jnp.concatenate([se_ref[...], te_ref[...]], axis=1)

    def bn(v, g, b):
        mu = jnp.mean(v, axis=0, keepdims=True)
        var = jnp.mean((v - mu) ** 2, axis=0, keepdims=True)
        return g * (v - mu) / jnp.sqrt(var + 1e-5) + b

    o = jnp.dot(x, Wfc1_ref[...].T, preferred_element_type=jnp.float32)
    o = _elu(bn(o + bfc1_ref[...], g1_ref[...], be1_ref[...]))
    o = jnp.dot(o, Wfc2_ref[...].T, preferred_element_type=jnp.float32)
    o = _elu(bn(o + bfc2_ref[...], g2_ref[...], be2_ref[...]))
    o = _elu(jnp.dot(o, Wfc3_ref[...].T, preferred_element_type=jnp.float32)
             + bfc3_ref[...])
    o_ref[...] = jnp.sum(o * Wout_ref[...], axis=1, keepdims=True) \
        + bout_ref[...]


def _mlp_head(stop_emb, temp_emb, Wfc1, bfc1, g1, be1, Wfc2, bfc2, g2, be2,
              Wfc3, bfc3, Wout, bout):
    return pl.pallas_call(
        _mlp_body,
        out_shape=jax.ShapeDtypeStruct((B, 1), jnp.float32),
    )(stop_emb, temp_emb, Wfc1, bfc1.reshape(1, -1), g1.reshape(1, -1),
      be1.reshape(1, -1), Wfc2, bfc2.reshape(1, -1), g2.reshape(1, -1),
      be2.reshape(1, -1), Wfc3, bfc3.reshape(1, -1), Wout,
      bout.reshape(1, -1))


def _gru_last(seq, lengths, Wih, Whh, bih, bhh):
    b = seq.shape[0]
    h0 = jnp.zeros((b, Whh.shape[1]), seq.dtype)

    def step(h, inp):
        x_t, t = inp
        gi = x_t @ Wih.T + bih
        gh = h @ Whh.T + bhh
        ir, iz, inn = jnp.split(gi, 3, axis=1)
        hr, hz, hn = jnp.split(gh, 3, axis=1)
        r = jax.nn.sigmoid(ir + hr)
        z = jax.nn.sigmoid(iz + hz)
        ng = jnp.tanh(inn + r * hn)
        h_new = (1.0 - z) * ng + z * h
        h = jnp.where((t < lengths)[:, None], h_new, h)
        return h, None

    hT, _ = jax.lax.scan(step, h0, (jnp.swapaxes(seq, 0, 1),
                                    jnp.arange(seq.shape[1])))
    return hT


def _gatv2_ref(x, src, dst, Wl, bl, Wr, br, att, bias, heads, outc, concat):
    n = x.shape[0]
    xl = (x @ Wl.T + bl).reshape(n, heads, outc)
    xr = (x @ Wr.T + br).reshape(n, heads, outc)
    e = jax.nn.leaky_relu(xl[src] + xr[dst], 0.2)
    logits = (e * att[None, :, :]).sum(-1)
    m = jax.ops.segment_max(logits, dst, num_segments=n)
    m = jnp.where(jnp.isfinite(m), m, 0.0)
    a = jnp.exp(logits - m[dst])
    denom = jax.ops.segment_sum(a, dst, num_segments=n)
    alpha = a / (denom[dst] + 1e-16)
    out = jax.ops.segment_sum(xl[src] * alpha[:, :, None], dst,
                              num_segments=n)
    out = out.reshape(n, heads * outc) if concat else out.mean(axis=1)
    return out + bias


def kernel(x_nodes, edge_index, seq_batch, lengths, stop_indices, W1l, b1l,
           W1r, b1r, att1, bias1, W2l, b2l, W2r, b2r, att2, bias2, Wih, Whh,
           bih, bhh, Wfc1, bfc1, g1, be1, Wfc2, bfc2, g2, be2, Wfc3, bfc3,
           Wout, bout):
    src, dst = edge_index[0], edge_index[1]
    zpad = jnp.zeros((EP - E,), jnp.int32)
    selfs = jnp.arange(NP, dtype=jnp.int32)
    src12 = jnp.concatenate([src, zpad])
    dst12 = jnp.concatenate([dst, zpad])
    src3 = jnp.concatenate([src, zpad, selfs])
    dst3 = jnp.concatenate([dst, zpad, selfs])
    xpad = jnp.pad(x_nodes, ((0, NP - N), (0, 0)))

    x1 = _gat_layer(xpad, W1l, b1l, W1r, b1r, att1, bias1, 4, 256, 128,
                    src12, dst12, src3, dst3)
    x2 = _gat_layer(x1, W2l, b2l, W2r, b2r, att2, bias2, 1, 128, 256,
                    src12, dst12, src3, dst3)

    stop_emb = _sc_row_gather(x2, stop_indices, 128)
    temp_emb = _gru_last(seq_batch, lengths, Wih, Whh, bih, bhh)
    return _mlp_head(stop_emb, temp_emb, Wfc1, bfc1, g1, be1, Wfc2, bfc2,
                     g2, be2, Wfc3, bfc3, Wout, bout)


# speedup vs baseline: 5.6466x; 1.1853x over previous
import functools

import jax
import jax.numpy as jnp
from jax import lax
from jax.experimental import pallas as pl
from jax.experimental.pallas import tpu as pltpu
from jax.experimental.pallas import tpu_sc as plsc


N, D, E = 10000, 128, 160000
B, T, F = 4096, 50, 16

NP = 10240
EP = 163840
EP3 = EP + NP

_SC_INFO = plsc.get_sparse_core_info()
_NC, _NS, _L = _SC_INFO.num_cores, _SC_INFO.num_subcores, _SC_INFO.num_lanes
_NW = _NC * _NS

_NEG = -3.0e38


def _iota16():
    return lax.iota(jnp.int32, 16)


def _f16(x, dtype=jnp.int32):
    return jnp.full((16,), x, dtype)


def _elu(x):
    return jnp.where(x > 0, x, jnp.exp(jnp.minimum(x, 0.0)) - 1.0)


def _scatter_combine(tab, tmpf, tmpi, dstv, vals, valid, op):
    li = _iota16()
    neutral = jnp.float32(_NEG if op == "max" else 0.0)

    def shift(vec, idx, tmp):
        tmp[...] = vec
        return plsc.load_gather(tmp, [idx])

    vld01 = jnp.where(valid, 1, 0)
    skeys, sperm = plsc.sort_key_val(dstv, li)
    svld = shift(vld01, sperm, tmpi)
    svs = []
    for v in vals:
        sv = shift(v, sperm, tmpf)
        svs.append(jnp.where(svld == 1, sv, neutral))
    for k in (1, 2, 4, 8):
        idx = jnp.maximum(li - k, 0)
        cond = jnp.logical_and(li >= k, shift(skeys, idx, tmpi) == skeys)
        for i, sv in enumerate(svs):
            sh = shift(sv, idx, tmpf)
            comb = jnp.maximum(sv, sh) if op == "max" else sv + sh
            svs[i] = jnp.where(cond, comb, sv)
        svld = jnp.where(cond, jnp.maximum(svld, shift(svld, idx, tmpi)),
                         svld)
    nxt = shift(skeys, jnp.minimum(li + 1, 15), tmpi)
    last = jnp.logical_or(li == 15, nxt != skeys)
    writers = jnp.logical_and(last, svld == 1)
    for h, sv in enumerate(svs):
        idxh = skeys + h * NP
        cur = plsc.load_gather(tab, [idxh], mask=writers)
        nv = jnp.maximum(cur, sv) if op == "max" else cur + sv
        plsc.store_scatter(tab, [idxh], nv, mask=writers)


def _init_vec(ref, n, value):
    def bd(i, _):
        ref[pl.ds(i * 16, 16)] = jnp.full((16,), value, ref.dtype)
        return 0

    lax.fori_loop(0, n // 16, bd, 0)


def _tile_dump(tab, out_hbm, wid, ht):
    pltpu.sync_copy(tab, out_hbm.at[pl.ds(wid * ht, ht)])


def _gat_sweep1(xsl, xsr, src_e, dst_e, att_flat, heads, dim, halves):
    C = dim // heads
    cb = 80 if halves == 2 else 160
    ept = EP // _NW
    nb = ept // cb
    ht = heads * NP
    rs = ht // _NS
    mesh = plsc.VectorSubcoreMesh(core_axis_name="c", subcore_axis_name="s")

    scratch = [
        pltpu.VMEM((dim,), jnp.float32),
        pltpu.VMEM((cb,), jnp.int32),
        pltpu.VMEM((cb,), jnp.int32),
        pltpu.VMEM((cb,), jnp.int32),
        pltpu.VMEM((cb,), jnp.int32),
        pltpu.VMEM((cb, 128), jnp.float32),
        pltpu.VMEM((cb, 128), jnp.float32),
        pltpu.VMEM((cb, 128), jnp.float32),
        pltpu.VMEM((cb, 128), jnp.float32),
        pltpu.VMEM((heads * cb,), jnp.float32),
        pltpu.VMEM((ht,), jnp.float32),
        pltpu.VMEM((16,), jnp.float32),
        pltpu.VMEM((16,), jnp.int32),
        pltpu.SemaphoreType.DMA,
    ]
    if halves == 1:
        scratch[6] = pltpu.VMEM((16,), jnp.float32)
        scratch[8] = pltpu.VMEM((16,), jnp.float32)

    @functools.partial(
        pl.kernel, mesh=mesh,
        compiler_params=pltpu.CompilerParams(needs_layout_passes=False),
        out_type=(jax.ShapeDtypeStruct((heads * EP,), jnp.float32),
                  jax.ShapeDtypeStruct((_NW * ht,), jnp.float32)),
        scratch_types=scratch,
    )
    def k(xsl_h, xsr_h, src_h, dst_h, att_h, logits_h, mpart_h,
          att_v, src_v, aux_v, dst_v, aux2_v, bl_lo, bl_hi, br_lo, br_hi,
          lbuf, mtab, tmpf, tmpi, sem):
        cid = lax.axis_index("c")
        sid = lax.axis_index("s")
        wid = sid * _NC + cid
        base = wid * ept
        pltpu.sync_copy(att_h, att_v)
        _init_vec(mtab, ht, _NEG)
        it16 = _iota16()

        def block(b, _):
            bb = base + b * cb
            h1 = pltpu.async_copy(src_h.at[pl.ds(bb, cb)], src_v, sem)
            h2 = pltpu.async_copy(dst_h.at[pl.ds(bb, cb)], dst_v, sem)
            h1.wait()
            h2.wait()
            if halves == 2:
                def off(j, _):
                    sl = pl.ds(j * 16, 16)
                    aux_v[sl] = src_v[sl] + NP
                    aux2_v[sl] = dst_v[sl] + NP
                    return 0
                lax.fori_loop(0, cb // 16, off, 0)
            g1 = pltpu.async_copy(xsl_h.at[src_v], bl_lo, sem)
            g2 = pltpu.async_copy(xsr_h.at[dst_v], br_lo, sem)
            if halves == 2:
                g3 = pltpu.async_copy(xsl_h.at[aux_v], bl_hi, sem)
                g4 = pltpu.async_copy(xsr_h.at[aux2_v], br_hi, sem)
            g1.wait()
            g2.wait()
            if halves == 2:
                g3.wait()
                g4.wait()

            def group(g, _):
                eb = g * 16
                dstv = dst_v[pl.ds(eb, 16)]
                valid = (bb + eb) + it16 < E
                ev = eb + it16
                vals = []
                for h in range(heads):
                    hf = (h * C) // 128
                    bl = bl_lo if hf == 0 else bl_hi
                    br = br_lo if hf == 0 else br_hi
                    f0 = (h * C) % 128

                    def cbody(c, acc, bl=bl, br=br, f0=f0, h=h):
                        fv = _f16(f0 + c)
                        s = (plsc.load_gather(bl, [ev, fv])
                             + plsc.load_gather(br, [ev, fv]))
                        lk = jnp.maximum(s, 0.2 * s)
                        av = plsc.load_gather(att_v, [_f16(h * C + c)])
                        return acc + av * lk

                    accv = lax.fori_loop(0, C, cbody,
                                         jnp.zeros((16,), jnp.float32))
                    lbuf[pl.ds(h * cb + eb, 16)] = accv
                    vals.append(accv)
                _scatter_combine(mtab, tmpf, tmpi, dstv, vals, valid, "max")
                return 0

            lax.fori_loop(0, cb // 16, group, 0)
            for h in range(heads):
                pltpu.sync_copy(lbuf.at[pl.ds(h * cb, cb)],
                                logits_h.at[pl.ds(h * EP + bb, cb)])
            return 0

        lax.fori_loop(0, nb, block, 0)
        _tile_dump(mtab, mpart_h, wid, ht)

    return k(xsl, xsr, src_e, dst_e, att_flat)


def _merge_a(m_part, lself, heads):
    ht = heads * NP
    rows = NP // _NW
    mesh = plsc.VectorSubcoreMesh(core_axis_name="c", subcore_axis_name="s")

    @functools.partial(
        pl.kernel, mesh=mesh,
        compiler_params=pltpu.CompilerParams(needs_layout_passes=False),
        out_type=(jax.ShapeDtypeStruct((ht,), jnp.float32),
                  jax.ShapeDtypeStruct((ht,), jnp.float32)),
        scratch_types=[
            pltpu.VMEM((rows * 8,), jnp.float32),
            pltpu.VMEM((rows,), jnp.float32),
            pltpu.VMEM((rows,), jnp.float32),
            pltpu.VMEM((rows,), jnp.float32),
            pltpu.VMEM((rows,), jnp.float32),
        ],
    )
    def k(mp_h, ls_h, mfin_h, aself_h, ls_v, m0, m1, mf, asb):
        cid = lax.axis_index("c")
        sid = lax.axis_index("s")
        wid = sid * _NC + cid
        r0 = wid * rows
        it16 = _iota16()
        pltpu.sync_copy(ls_h.at[pl.ds(r0 * 8, rows * 8)], ls_v)
        for h in range(heads):
            pltpu.sync_copy(mp_h.at[pl.ds(h * NP + r0, rows)], m0)
            for w in range(1, _NW):
                pltpu.sync_copy(mp_h.at[pl.ds(w * ht + h * NP + r0, rows)],
                                m1)

                def rd(g, _):
                    sl = pl.ds(g * 16, 16)
                    m0[sl] = jnp.maximum(m0[sl], m1[sl])
                    return 0

                lax.fori_loop(0, rows // 16, rd, 0)

            def bd(g, _, h=h):
                sl = pl.ds(g * 16, 16)
                lsv = plsc.load_gather(ls_v, [(g * 16 + it16) * 8 + h])
                mv = jnp.maximum(m0[sl], lsv)
                mf[sl] = mv
                asb[sl] = jnp.exp(lsv - mv)
                return 0

            lax.fori_loop(0, rows // 16, bd, 0)
            pltpu.sync_copy(mf, mfin_h.at[pl.ds(h * NP + r0, rows)])
            pltpu.sync_copy(asb, aself_h.at[pl.ds(h * NP + r0, rows)])

    return k(m_part, lself)


def _gat_sweep2(logits, dst_e, m_fin, heads):
    cb = 320
    ept = EP // _NW
    nb = ept // cb
    ht = heads * NP
    rs = ht // _NS
    mesh = plsc.VectorSubcoreMesh(core_axis_name="c", subcore_axis_name="s")

    @functools.partial(
        pl.kernel, mesh=mesh,
        compiler_params=pltpu.CompilerParams(needs_layout_passes=False),
        out_type=(jax.ShapeDtypeStruct((heads * EP,), jnp.float32),
                  jax.ShapeDtypeStruct((_NW * ht,), jnp.float32)),
        scratch_types=[
            pltpu.VMEM((ht,), jnp.float32),
            pltpu.VMEM((ht,), jnp.float32),
            pltpu.VMEM((cb,), jnp.int32),
            pltpu.VMEM((heads * cb,), jnp.float32),
            pltpu.VMEM((heads * cb,), jnp.float32),
            pltpu.VMEM((16,), jnp.float32),
            pltpu.VMEM((16,), jnp.int32),
            pltpu.SemaphoreType.DMA,
        ],
    )
    def k(lg_h, dst_h, mfin_h, a_h, dpart_h,
          m_v, den, dst_v, lbuf, abuf, tmpf, tmpi, sem):
        cid = lax.axis_index("c")
        sid = lax.axis_index("s")
        wid = sid * _NC + cid
        base = wid * ept
        it16 = _iota16()
        pltpu.sync_copy(mfin_h, m_v)
        _init_vec(den, ht, 0.0)

        def block(b, _):
            bb = base + b * cb
            hh = [pltpu.async_copy(dst_h.at[pl.ds(bb, cb)], dst_v, sem)]
            for h in range(heads):
                hh.append(pltpu.async_copy(lg_h.at[pl.ds(h * EP + bb, cb)],
                                           lbuf.at[pl.ds(h * cb, cb)], sem))
            for x in hh:
                x.wait()

            def group(g, _):
                eb = g * 16
                dstv = dst_v[pl.ds(eb, 16)]
                valid = (bb + eb) + it16 < E
                vals = []
                for h in range(heads):
                    mg = plsc.load_gather(m_v, [dstv + h * NP])
                    a = jnp.exp(lbuf[pl.ds(h * cb + eb, 16)] - mg)
                    abuf[pl.ds(h * cb + eb, 16)] = a
                    vals.append(a)
                _scatter_combine(den, tmpf, tmpi, dstv, vals, valid, "add")
                return 0

            lax.fori_loop(0, cb // 16, group, 0)
            for h in range(heads):
                pltpu.sync_copy(abuf.at[pl.ds(h * cb, cb)],
                                a_h.at[pl.ds(h * EP + bb, cb)])
            return 0

        lax.fori_loop(0, nb, block, 0)
        _tile_dump(den, dpart_h, wid, ht)

    return k(logits, dst_e, m_fin)


def _alpha_sweep(a_full, dst_e, dinv, heads):
    cb = 320
    ept = EP3 // _NW
    nb = ept // cb
    ht = heads * NP
    mesh = plsc.VectorSubcoreMesh(core_axis_name="c", subcore_axis_name="s")

    @functools.partial(
        pl.kernel, mesh=mesh,
        compiler_params=pltpu.CompilerParams(needs_layout_passes=False),
        out_type=jax.ShapeDtypeStruct((heads * EP3,), jnp.float32),
        scratch_types=[
            pltpu.VMEM((ht,), jnp.float32),
            pltpu.VMEM((cb,), jnp.int32),
            pltpu.VMEM((heads * cb,), jnp.float32),
            pltpu.VMEM((heads * cb,), jnp.float32),
            pltpu.SemaphoreType.DMA,
        ],
    )
    def k(af_h, dst_h, dinv_h, alpha_h, dv, dst_v, abuf, alb, sem):
        cid = lax.axis_index("c")
        sid = lax.axis_index("s")
        wid = sid * _NC + cid
        base = wid * ept
        it16 = _iota16()
        pltpu.sync_copy(dinv_h, dv)

        def block(b, _):
            bb = base + b * cb
            hh = [pltpu.async_copy(dst_h.at[pl.ds(bb, cb)], dst_v, sem)]
            for h in range(heads):
                hh.append(pltpu.async_copy(af_h.at[pl.ds(h * EP3 + bb, cb)],
                                           abuf.at[pl.ds(h * cb, cb)], sem))
            for x in hh:
                x.wait()

            def group(g, _):
                eb = g * 16
                dstv = dst_v[pl.ds(eb, 16)]
                gid = (bb + eb) + it16
                valid = jnp.logical_or(gid < E, gid >= EP)
                for h in range(heads):
                    di = plsc.load_gather(dv, [dstv + h * NP])
                    al = abuf[pl.ds(h * cb + eb, 16)] * di
                    alb[pl.ds(h * cb + eb, 16)] = jnp.where(valid, al, 0.0)
                return 0

            lax.fori_loop(0, cb // 16, group, 0)
            for h in range(heads):
                pltpu.sync_copy(alb.at[pl.ds(h * cb, cb)],
                                alpha_h.at[pl.ds(h * EP3 + bb, cb)])
            return 0

        lax.fori_loop(0, nb, block, 0)

    return k(a_full, dst_e, dinv)


def _gat_sweep3(table, src_e, dst_e, alpha, heads, feature_split):
    cb = 160
    if feature_split:
        ept = EP3 // _NS
    else:
        ept = EP3 // _NW
    nb = ept // cb
    mesh = plsc.VectorSubcoreMesh(core_axis_name="c", subcore_axis_name="s")

    @functools.partial(
        pl.kernel, mesh=mesh,
        compiler_params=pltpu.CompilerParams(needs_layout_passes=False),
        out_type=jax.ShapeDtypeStruct((2 * NP, 128), jnp.float32),
        scratch_types=[
            pltpu.VMEM((cb,), jnp.int32),
            pltpu.VMEM((cb,), jnp.int32),
            pltpu.VMEM((cb,), jnp.int32),
            pltpu.VMEM((cb, 128), jnp.float32),
            pltpu.VMEM((heads * cb,), jnp.float32),
            pltpu.VMEM_SHARED((NP, 128), jnp.float32),
            pltpu.SemaphoreType.DMA,
        ],
    )
    def k(tab_h, src_h, dst_h, alpha_h, out_h,
          src_v, tix, dst_v, rows, alb, spm, sem):
        cid = lax.axis_index("c")
        sid = lax.axis_index("s")

        def ze(e, _):
            for ch in range(8):
                rows[e, pl.ds(ch * 16, 16)] = jnp.zeros((16,), jnp.float32)
            return 0
        lax.fori_loop(0, cb, ze, 0)
        for q in range(NP // _NS // cb):
            pltpu.sync_copy(rows, spm.at[pl.ds(sid * (NP // _NS) + q * cb,
                                               cb)])
        plsc.subcore_barrier()

        if feature_split:
            tbase = sid * ept
        else:
            tbase = (sid * _NC + cid) * ept

        def block(b, _):
            eb = tbase + b * cb
            hh = [pltpu.async_copy(src_h.at[pl.ds(eb, cb)], src_v, sem),
                  pltpu.async_copy(dst_h.at[pl.ds(eb, cb)], dst_v, sem)]
            for h in range(heads):
                hh.append(
                    pltpu.async_copy(alpha_h.at[pl.ds(h * EP3 + eb, cb)],
                                     alb.at[pl.ds(h * cb, cb)], sem))
            for x in hh:
                x.wait()
            if feature_split:
                def off(j, _):
                    sl = pl.ds(j * 16, 16)
                    tix[sl] = src_v[sl] + cid * NP
                    return 0
                lax.fori_loop(0, cb // 16, off, 0)
                pltpu.sync_copy(tab_h.at[tix], rows)
            else:
                pltpu.sync_copy(tab_h.at[src_v], rows)

            nheads_half = 2 if (feature_split and heads == 4) else 1
            cph = 8 // nheads_half

            def scale(e, _):
                for hh in range(nheads_half):
                    if feature_split and heads == 4:
                        hrow = cid * 2 + hh
                    else:
                        hrow = 0
                    aspl = plsc.load_gather(alb, [_f16(hrow * cb + e)])
                    for ch in range(hh * cph, (hh + 1) * cph):
                        sl = pl.ds(ch * 16, 16)
                        rows[e, sl] = rows[e, sl] * aspl
                return 0

            lax.fori_loop(0, cb, scale, 0)
            pltpu.sync_copy(rows, spm.at[dst_v], add=True)
            return 0

        lax.fori_loop(0, nb, block, 0)
        plsc.subcore_barrier()
        o = sid * (NP // _NS)
        pltpu.sync_copy(spm.at[pl.ds(o, NP // _NS)],
                        out_h.at[pl.ds(cid * NP + o, NP // _NS)])

    return k(table, src_e, dst_e, alpha)


def _proj_body(x_ref, wl_ref, bl_ref, wr_ref, br_ref, attm_ref,
               xl_ref, xr_ref, ls_ref):
    xb = x_ref[...]
    xl = jnp.dot(xb, wl_ref[...].T, preferred_element_type=jnp.float32) \
        + bl_ref[...]
    xr = jnp.dot(xb, wr_ref[...].T, preferred_element_type=jnp.float32) \
        + br_ref[...]
    s = xl + xr
    lk = jnp.maximum(s, 0.2 * s)
    xl_ref[...] = xl
    xr_ref[...] = xr
    ls_ref[...] = jnp.dot(lk, attm_ref[...],
                          preferred_element_type=jnp.float32)


def _proj(x, Wl, bl, Wr, br, attM, din, dim):
    nblk = 16
    rows = NP // nblk
    return pl.pallas_call(
        _proj_body,
        grid=(nblk,),
        in_specs=[
            pl.BlockSpec((rows, din), lambda i: (i, 0)),
            pl.BlockSpec((dim, din), lambda i: (0, 0)),
            pl.BlockSpec((1, dim), lambda i: (0, 0)),
            pl.BlockSpec((dim, din), lambda i: (0, 0)),
            pl.BlockSpec((1, dim), lambda i: (0, 0)),
            pl.BlockSpec((dim, 8), lambda i: (0, 0)),
        ],
        out_specs=[
            pl.BlockSpec((rows, dim), lambda i: (i, 0)),
            pl.BlockSpec((rows, dim), lambda i: (i, 0)),
            pl.BlockSpec((rows, 8), lambda i: (i, 0)),
        ],
        out_shape=[
            jax.ShapeDtypeStruct((NP, dim), jnp.float32),
            jax.ShapeDtypeStruct((NP, dim), jnp.float32),
            jax.ShapeDtypeStruct((NP, 8), jnp.float32),
        ],
    )(x, Wl, bl.reshape(1, -1), Wr, br.reshape(1, -1), attM)


def _merge_b(d_part, a_self, heads):
    ht = heads * NP

    def body(d_ref, a_ref, o_ref):
        o_ref[...] = 1.0 / (jnp.sum(d_ref[...], axis=0) + a_ref[...] + 1e-16)

    out = pl.pallas_call(
        body,
        out_shape=jax.ShapeDtypeStruct((ht // 128, 128), jnp.float32),
    )(d_part.reshape(_NW, ht // 128, 128), a_self.reshape(ht // 128, 128))
    return out.reshape(ht)


def _post1(acc, bias):
    def body(a0_ref, a1_ref, b_ref, o_ref):
        o_ref[:, 0:128] = _elu(a0_ref[...] + b_ref[:, 0:128])
        o_ref[:, 128:256] = _elu(a1_ref[...] + b_ref[:, 128:256])

    nblk = 16
    rows = NP // nblk
    return pl.pallas_call(
        body,
        grid=(nblk,),
        in_specs=[
            pl.BlockSpec((rows, 128), lambda i: (i, 0)),
            pl.BlockSpec((rows, 128), lambda i: (i, 0)),
            pl.BlockSpec((1, 256), lambda i: (0, 0)),
        ],
        out_specs=pl.BlockSpec((rows, 256), lambda i: (i, 0)),
        out_shape=jax.ShapeDtypeStruct((NP, 256), jnp.float32),
    )(acc[0], acc[1], bias.reshape(1, -1))


def _post2(acc, bias):
    def body(a0_ref, a1_ref, b_ref, o_ref):
        o_ref[...] = _elu(a0_ref[...] + a1_ref[...] + b_ref[...])

    nblk = 16
    rows = NP // nblk
    return pl.pallas_call(
        body,
        grid=(nblk,),
        in_specs=[
            pl.BlockSpec((rows, 128), lambda i: (i, 0)),
            pl.BlockSpec((rows, 128), lambda i: (i, 0)),
            pl.BlockSpec((1, 128), lambda i: (0, 0)),
        ],
        out_specs=pl.BlockSpec((rows, 128), lambda i: (i, 0)),
        out_shape=jax.ShapeDtypeStruct((NP, 128), jnp.float32),
    )(acc[0], acc[1], bias.reshape(1, -1))


def _gat_layer(x, Wl, bl, Wr, br, att, bias, heads, dim, din,
               src12, dst12, src3, dst3):
    C = dim // heads
    attM = jnp.zeros((dim, 8), jnp.float32).at[
        jnp.arange(dim), jnp.arange(dim) // C].set(att.reshape(-1))
    xl, xr, ls = _proj(x, Wl, bl, Wr, br, attM, din, dim)
    if dim == 256:
        xsl = jnp.concatenate([xl[:, :128], xl[:, 128:]], axis=0)
        xsr = jnp.concatenate([xr[:, :128], xr[:, 128:]], axis=0)
        logits, m_part = _gat_sweep1(xsl, xsr, src12, dst12,
                                     att.reshape(-1), heads, dim, 2)
    else:
        xsl, xsr = xl, xr
        logits, m_part = _gat_sweep1(xsl, xsr, src12, dst12,
                                     att.reshape(-1), heads, dim, 1)
    m_fin, a_self = _merge_a(m_part, ls.reshape(-1), heads)
    a_edge, d_part = _gat_sweep2(logits, dst12, m_fin, heads)
    dinv = _merge_b(d_part, a_self, heads)
    a_full = jnp.concatenate([a_edge.reshape(heads, EP),
                              a_self.reshape(heads, NP)], axis=1).reshape(-1)
    alpha = _alpha_sweep(a_full, dst3, dinv, heads)
    acc = _gat_sweep3(xsl, src3, dst3, alpha, heads,
                      feature_split=(dim == 256))
    acc = (acc[:NP], acc[NP:])
    if dim == 256:
        return _post1(acc, bias)
    return _post2(acc, bias)


def _sc_row_gather(table, idx, cols):
    nidx = idx.shape[0]
    per_w = nidx // _NW
    mesh = plsc.VectorSubcoreMesh(core_axis_name="c", subcore_axis_name="s")

    @functools.partial(
        pl.kernel, mesh=mesh,
        compiler_params=pltpu.CompilerParams(needs_layout_passes=False),
        out_type=jax.ShapeDtypeStruct((nidx, cols), jnp.float32),
        scratch_types=[
            pltpu.VMEM((per_w,), jnp.int32),
            pltpu.VMEM((per_w, cols), jnp.float32),
            pltpu.SemaphoreType.DMA,
        ],
    )
    def k(table_hbm, idx_hbm, out_hbm, idx_v, rows_v, sem):
        wid = lax.axis_index("s") * _NC + lax.axis_index("c")
        base = wid * per_w
        pltpu.sync_copy(idx_hbm.at[pl.ds(base, per_w)], idx_v)
        pltpu.async_copy(table_hbm.at[idx_v], rows_v, sem).wait()
        pltpu.sync_copy(rows_v, out_hbm.at[pl.ds(base, per_w)])

    return k(table, idx)


def _mlp_body(se_ref, te_ref, Wfc1_ref, bfc1_ref, g1_ref, be1_ref, Wfc2_ref,
              bfc2_ref, g2_ref, be2_ref, Wfc3_ref, bfc3_ref, Wout_ref,
              bout_ref, o_ref):
    x = jnp.concatenate([se_ref[...], te_ref[...]], axis=1)

    def bn(v, g, b):
        mu = jnp.mean(v, axis=0, keepdims=True)
        var = jnp.mean((v - mu) ** 2, axis=0, keepdims=True)
        return g * (v - mu) / jnp.sqrt(var + 1e-5) + b

    o = jnp.dot(x, Wfc1_ref[...].T, preferred_element_type=jnp.float32)
    o = _elu(bn(o + bfc1_ref[...], g1_ref[...], be1_ref[...]))
    o = jnp.dot(o, Wfc2_ref[...].T, preferred_element_type=jnp.float32)
    o = _elu(bn(o + bfc2_ref[...], g2_ref[...], be2_ref[...]))
    o = _elu(jnp.dot(o, Wfc3_ref[...].T, preferred_element_type=jnp.float32)
             + bfc3_ref[...])
    o_ref[...] = jnp.sum(o * Wout_ref[...], axis=1, keepdims=True) \
        + bout_ref[...]


def _mlp_head(stop_emb, temp_emb, Wfc1, bfc1, g1, be1, Wfc2, bfc2, g2, be2,
              Wfc3, bfc3, Wout, bout):
    return pl.pallas_call(
        _mlp_body,
        out_shape=jax.ShapeDtypeStruct((B, 1), jnp.float32),
    )(stop_emb, temp_emb, Wfc1, bfc1.reshape(1, -1), g1.reshape(1, -1),
      be1.reshape(1, -1), Wfc2, bfc2.reshape(1, -1), g2.reshape(1, -1),
      be2.reshape(1, -1), Wfc3, bfc3.reshape(1, -1), Wout,
      bout.reshape(1, -1))


def _gru_last(seq, lengths, Wih, Whh, bih, bhh):
    b = seq.shape[0]
    h0 = jnp.zeros((b, Whh.shape[1]), seq.dtype)

    def step(h, inp):
        x_t, t = inp
        gi = x_t @ Wih.T + bih
        gh = h @ Whh.T + bhh
        ir, iz, inn = jnp.split(gi, 3, axis=1)
        hr, hz, hn = jnp.split(gh, 3, axis=1)
        r = jax.nn.sigmoid(ir + hr)
        z = jax.nn.sigmoid(iz + hz)
        ng = jnp.tanh(inn + r * hn)
        h_new = (1.0 - z) * ng + z * h
        h = jnp.where((t < lengths)[:, None], h_new, h)
        return h, None

    hT, _ = jax.lax.scan(step, h0, (jnp.swapaxes(seq, 0, 1),
                                    jnp.arange(seq.shape[1])))
    return hT


def _gatv2_ref(x, src, dst, Wl, bl, Wr, br, att, bias, heads, outc, concat):
    n = x.shape[0]
    xl = (x @ Wl.T + bl).reshape(n, heads, outc)
    xr = (x @ Wr.T + br).reshape(n, heads, outc)
    e = jax.nn.leaky_relu(xl[src] + xr[dst], 0.2)
    logits = (e * att[None, :, :]).sum(-1)
    m = jax.ops.segment_max(logits, dst, num_segments=n)
    m = jnp.where(jnp.isfinite(m), m, 0.0)
    a = jnp.exp(logits - m[dst])
    denom = jax.ops.segment_sum(a, dst, num_segments=n)
    alpha = a / (denom[dst] + 1e-16)
    out = jax.ops.segment_sum(xl[src] * alpha[:, :, None], dst,
                              num_segments=n)
    out = out.reshape(n, heads * outc) if concat else out.mean(axis=1)
    return out + bias


def kernel(x_nodes, edge_index, seq_batch, lengths, stop_indices, W1l, b1l,
           W1r, b1r, att1, bias1, W2l, b2l, W2r, b2r, att2, bias2, Wih, Whh,
           bih, bhh, Wfc1, bfc1, g1, be1, Wfc2, bfc2, g2, be2, Wfc3, bfc3,
           Wout, bout):
    src, dst = edge_index[0], edge_index[1]
    zpad = jnp.zeros((EP - E,), jnp.int32)
    selfs = jnp.arange(NP, dtype=jnp.int32)
    src12 = jnp.concatenate([src, zpad])
    dst12 = jnp.concatenate([dst, zpad])
    src3 = jnp.concatenate([src, zpad, selfs])
    dst3 = jnp.concatenate([dst, zpad, selfs])
    xpad = jnp.pad(x_nodes, ((0, NP - N), (0, 0)))

    x1 = _gat_layer(xpad, W1l, b1l, W1r, b1r, att1, bias1, 4, 256, 128,
                    src12, dst12, src3, dst3)
    x2 = _gat_layer(x1, W2l, b2l, W2r, b2r, att2, bias2, 1, 128, 256,
                    src12, dst12, src3, dst3)

    stop_emb = _sc_row_gather(x2, stop_indices, 128)
    temp_emb = _gru_last(seq_batch, lengths, Wih, Whh, bih, bhh)
    return _mlp_head(stop_emb, temp_emb, Wfc1, bfc1, g1, be1, Wfc2, bfc2,
                     g2, be2, Wfc3, bfc3, Wout, bout)
